# baseline (speedup 1.0000x reference)
import functools

import jax
import jax.numpy as jnp
from jax import lax
from jax.experimental import pallas as pl
from jax.experimental.pallas import tpu as pltpu
from jax.experimental.pallas import tpu_sc as plsc

TOPK = 128
EPS = 0.001
LT = 8


def _tc_body(q_ref, k_ref, s_ref, cut_ref, colsum):
    ph = pl.program_id(1)

    @pl.when(ph == 0)
    def _colsum_phase():
        qp = jax.nn.relu(q_ref[0]) + EPS
        colsum[...] = jax.lax.dot_general(
            jnp.full((1, qp.shape[0]), 1.0, jnp.float32), qp,
            (((1,), (0,)), ((), ())),
            preferred_element_type=jnp.float32,
        )

    @pl.when(ph == 1)
    def _score_phase():
        kp = jax.nn.relu(k_ref[0]) + EPS
        sall = jax.lax.dot_general(
            colsum[...], kp, (((1,), (1,)), ((), ())),
            preferred_element_type=jnp.float32,
        )

        def step(_, lohi):
            lo, hi = lohi
            mid = lo + (hi - lo) // 2
            mid_f = jax.lax.bitcast_convert_type(mid, jnp.float32)
            cnt = jnp.sum((sall > mid_f).astype(jnp.int32))
            take = cnt <= TOPK
            return (
                jnp.where(take, lo, mid + 1),
                jnp.where(take, mid, hi),
            )

        lo, _ = jax.lax.fori_loop(
            0, 31, step, (jnp.int32(0), jnp.int32(0x7F800000))
        )
        cut_f = jax.lax.bitcast_convert_type(lo, jnp.float32)
        s_ref[0] = sall
        cut_ref[0, 0, :] = jnp.full((16,), cut_f, jnp.float32)


def _tc_scores(queries, keys):
    B, L, D = queries.shape
    blk = (1, L, D)

    def q_map(b, ph):
        return (b, 0, 0)

    def k_map(b, ph):
        return (b, 0, 0)

    def o_map(b, ph):
        return (b, 0, 0)

    return pl.pallas_call(
        _tc_body,
        grid=(B, 2),
        in_specs=[
            pl.BlockSpec(blk, q_map),
            pl.BlockSpec(blk, k_map),
        ],
        out_specs=[
            pl.BlockSpec((1, 1, L), o_map),
            pl.BlockSpec((1, 1, 16), o_map),
        ],
        out_shape=[
            jax.ShapeDtypeStruct((B, 1, L), jnp.float32),
            jax.ShapeDtypeStruct((B, 1, 16), jnp.float32),
        ],
        scratch_shapes=[
            pltpu.VMEM((1, D), jnp.float32),
        ],
        compiler_params=pltpu.CompilerParams(
            dimension_semantics=("arbitrary", "arbitrary"),
        ),
    )(queries, keys)


_NTILE = 16
_ROWS_PER_SC = 16


def _take16(vec, idx):
    return lax.gather(
        vec,
        idx[:, None],
        lax.GatherDimensionNumbers(
            offset_dims=(), collapsed_slice_dims=(0,), start_index_map=(0,)
        ),
        (1,),
        mode=lax.GatherScatterMode.PROMISE_IN_BOUNDS,
    )


def _sc_body(L, D, B, v_hbm, s_hbm, cut_hbm, out_hbm,
             scbuf, cutv, cutl, idxbuf, mbuf, idx_v, m_v, rows_v,
             spm_idx, spm_m, gsem):
    c = lax.axis_index("c")
    s = lax.axis_index("s")
    batches_per_core = B // 2

    @pl.when(s < batches_per_core)
    def _compact():
        b = c * batches_per_core + s
        pltpu.sync_copy(s_hbm.at[b], scbuf)
        pltpu.sync_copy(cut_hbm.at[b], cutv)
        cutf = cutv[...]
        zeros16 = jnp.zeros((16,), jnp.int32)

        def _findcut(i, _):
            sv = scbuf[pl.ds(i * 16, 16)]
            lvec = lax.iota(jnp.int32, 16) + i * 16
            plsc.store_scatter(cutl, [zeros16], lvec, mask=sv == cutf)
            return 0

        lax.fori_loop(0, L // 16, _findcut, 0)
        pad = _take16(cutl[...], zeros16) + b * L

        for j in range(TOPK // 16):
            idxbuf[pl.ds(j * 16, 16)] = pad
            mbuf[pl.ds(j * 16, 16)] = jnp.zeros((16,), jnp.float32)

        def _scan(i, carry):
            sv = scbuf[pl.ds(i * 16, 16)]
            keep = sv > cutf
            mi = keep.astype(jnp.int32)
            incl = plsc.cumsum(mi)
            pos = incl - mi + carry
            lvec = lax.iota(jnp.int32, 16) + i * 16 + b * L
            plsc.store_scatter(idxbuf, [pos], lvec, mask=keep)
            plsc.store_scatter(
                mbuf, [pos], jnp.ones((16,), jnp.float32), mask=keep
            )
            return carry + incl[15]

        lax.fori_loop(0, L // 16, _scan, jnp.int32(0))
        pltpu.sync_copy(idxbuf, spm_idx.at[s])
        pltpu.sync_copy(mbuf, spm_m.at[s])

    plsc.subcore_barrier()

    bb = s // (_NTILE // batches_per_core)
    off = (s % (_NTILE // batches_per_core)) * _ROWS_PER_SC
    pltpu.sync_copy(spm_idx.at[bb, pl.ds(off, _ROWS_PER_SC)], idx_v)
    pltpu.sync_copy(spm_m.at[bb, pl.ds(off, _ROWS_PER_SC)], m_v)
    pltpu.make_async_copy(v_hbm.at[idx_v], rows_v, gsem).start()
    pltpu.make_async_copy(v_hbm.at[idx_v], rows_v, gsem).wait()
    mv = m_v[...]

    def _scale_row(r, _):
        sc = _take16(mv, jnp.full((16,), r, jnp.int32))

        def _scale_chunk(d, _):
            rows_v[r, pl.ds(d * 16, 16)] = rows_v[r, pl.ds(d * 16, 16)] * sc
            return 0

        lax.fori_loop(0, D // 16, _scale_chunk, 0)
        return 0

    lax.fori_loop(0, _ROWS_PER_SC, _scale_row, 0)
    pltpu.make_async_copy(rows_v, out_hbm.at[idx_v], gsem).start()
    pltpu.make_async_copy(rows_v, out_hbm.at[idx_v], gsem).wait()


def _sc_apply(v_flat, scores_flat, cut, out_ref, L, D):
    B = scores_flat.shape[0]
    mesh = plsc.VectorSubcoreMesh(core_axis_name="c", subcore_axis_name="s")
    body = functools.partial(_sc_body, L, D, B)
    run = pl.kernel(
        body,
        out_type=(),
        mesh=mesh,
        scratch_types=[
            pltpu.VMEM((L,), jnp.float32),
            pltpu.VMEM((16,), jnp.float32),
            pltpu.VMEM((16,), jnp.int32),
            pltpu.VMEM((TOPK,), jnp.int32),
            pltpu.VMEM((TOPK,), jnp.float32),
            pltpu.VMEM((_ROWS_PER_SC,), jnp.int32),
            pltpu.VMEM((_ROWS_PER_SC,), jnp.float32),
            pltpu.VMEM((_ROWS_PER_SC, D), jnp.float32),
            pltpu.VMEM_SHARED((2, TOPK), jnp.int32),
            pltpu.VMEM_SHARED((2, TOPK), jnp.float32),
            pltpu.SemaphoreType.DMA,
        ],
        compiler_params=pltpu.CompilerParams(needs_layout_passes=False),
    )
    run(v_flat, scores_flat, cut, out_ref)


@jax.jit
def kernel(queries, keys, values):
    B, L, D = queries.shape
    scores, cut = _tc_scores(queries, keys)
    out_ref = jax.new_ref(jnp.zeros((B * L, D), jnp.float32))
    _sc_apply(
        values.reshape(B * L, D), scores.reshape(B, L), cut.reshape(B, 16),
        out_ref, L, D,
    )
    return out_ref[...].reshape(B, L, D)


# speedup vs baseline: 1.0421x; 1.0421x over previous
import functools

import jax
import jax.numpy as jnp
from jax import lax
from jax.experimental import pallas as pl
from jax.experimental.pallas import tpu as pltpu
from jax.experimental.pallas import tpu_sc as plsc

TOPK = 128
EPS = 0.001
LT = 8


def _tc_body(q_ref, k_ref, z_ref, s_ref, cut_ref, colsum):
    z_ref[...] = jnp.zeros_like(z_ref)
    ph = pl.program_id(1)

    @pl.when(ph == 0)
    def _colsum_phase():
        qp = jax.nn.relu(q_ref[0]) + EPS
        colsum[...] = jax.lax.dot_general(
            jnp.full((1, qp.shape[0]), 1.0, jnp.float32), qp,
            (((1,), (0,)), ((), ())),
            preferred_element_type=jnp.float32,
        )

    @pl.when(ph == 1)
    def _score_phase():
        kp = jax.nn.relu(k_ref[0]) + EPS
        sall = jax.lax.dot_general(
            colsum[...], kp, (((1,), (1,)), ((), ())),
            preferred_element_type=jnp.float32,
        )

        def step(_, lohi):
            lo, hi = lohi
            mid = lo + (hi - lo) // 2
            mid_f = jax.lax.bitcast_convert_type(mid, jnp.float32)
            cnt = jnp.sum((sall > mid_f).astype(jnp.int32))
            take = cnt <= TOPK
            return (
                jnp.where(take, lo, mid + 1),
                jnp.where(take, mid, hi),
            )

        lo, _ = jax.lax.fori_loop(
            0, 31, step, (jnp.int32(0), jnp.int32(0x7F800000))
        )
        cut_f = jax.lax.bitcast_convert_type(lo, jnp.float32)
        s_ref[0] = sall
        cut_ref[0, 0, :] = jnp.full((16,), cut_f, jnp.float32)


def _tc_scores(queries, keys):
    B, L, D = queries.shape
    blk = (1, L, D)

    def q_map(b, ph):
        return (b, 0, 0)

    def k_map(b, ph):
        return (b, 0, 0)

    def o_map(b, ph):
        return (b, 0, 0)

    def z_map(b, ph):
        return (b, ph, 0)

    return pl.pallas_call(
        _tc_body,
        grid=(B, 2),
        in_specs=[
            pl.BlockSpec(blk, q_map),
            pl.BlockSpec(blk, k_map),
        ],
        out_specs=[
            pl.BlockSpec((1, L // 2, D), z_map),
            pl.BlockSpec((1, 1, L), o_map),
            pl.BlockSpec((1, 1, 16), o_map),
        ],
        out_shape=[
            jax.ShapeDtypeStruct((B, L, D), jnp.float32),
            jax.ShapeDtypeStruct((B, 1, L), jnp.float32),
            jax.ShapeDtypeStruct((B, 1, 16), jnp.float32),
        ],
        scratch_shapes=[
            pltpu.VMEM((1, D), jnp.float32),
        ],
        compiler_params=pltpu.CompilerParams(
            dimension_semantics=("arbitrary", "arbitrary"),
        ),
    )(queries, keys)


_NTILE = 16
_ROWS_PER_SC = 16


def _take16(vec, idx):
    return lax.gather(
        vec,
        idx[:, None],
        lax.GatherDimensionNumbers(
            offset_dims=(), collapsed_slice_dims=(0,), start_index_map=(0,)
        ),
        (1,),
        mode=lax.GatherScatterMode.PROMISE_IN_BOUNDS,
    )


def _sc_body(L, D, B, v_hbm, s_hbm, cut_hbm, out_hbm,
             scbuf, cutv, cutl, idxbuf, mbuf, idx_v, m_v, rows_v,
             spm_idx, spm_m, gsem):
    c = lax.axis_index("c")
    s = lax.axis_index("s")
    batches_per_core = B // 2

    @pl.when(s < batches_per_core)
    def _compact():
        b = c * batches_per_core + s
        pltpu.sync_copy(s_hbm.at[b], scbuf)
        pltpu.sync_copy(cut_hbm.at[b], cutv)
        cutf = cutv[...]
        zeros16 = jnp.zeros((16,), jnp.int32)

        def _findcut(i, _):
            sv = scbuf[pl.ds(i * 16, 16)]
            lvec = lax.iota(jnp.int32, 16) + i * 16
            plsc.store_scatter(cutl, [zeros16], lvec, mask=sv == cutf)
            return 0

        lax.fori_loop(0, L // 16, _findcut, 0)
        pad = _take16(cutl[...], zeros16) + b * L

        for j in range(TOPK // 16):
            idxbuf[pl.ds(j * 16, 16)] = pad
            mbuf[pl.ds(j * 16, 16)] = jnp.zeros((16,), jnp.float32)

        def _scan(i, carry):
            sv = scbuf[pl.ds(i * 16, 16)]
            keep = sv > cutf
            mi = keep.astype(jnp.int32)
            incl = plsc.cumsum(mi)
            pos = incl - mi + carry
            lvec = lax.iota(jnp.int32, 16) + i * 16 + b * L
            plsc.store_scatter(idxbuf, [pos], lvec, mask=keep)
            plsc.store_scatter(
                mbuf, [pos], jnp.ones((16,), jnp.float32), mask=keep
            )
            return carry + incl[15]

        lax.fori_loop(0, L // 16, _scan, jnp.int32(0))
        pltpu.sync_copy(idxbuf, spm_idx.at[s])
        pltpu.sync_copy(mbuf, spm_m.at[s])

    plsc.subcore_barrier()

    bb = s // (_NTILE // batches_per_core)
    off = (s % (_NTILE // batches_per_core)) * _ROWS_PER_SC
    pltpu.sync_copy(spm_idx.at[bb, pl.ds(off, _ROWS_PER_SC)], idx_v)
    pltpu.sync_copy(spm_m.at[bb, pl.ds(off, _ROWS_PER_SC)], m_v)
    pltpu.make_async_copy(v_hbm.at[idx_v], rows_v, gsem).start()
    pltpu.make_async_copy(v_hbm.at[idx_v], rows_v, gsem).wait()
    mv = m_v[...]

    def _scale_row(r, _):
        sc = _take16(mv, jnp.full((16,), r, jnp.int32))

        def _scale_chunk(d, _):
            rows_v[r, pl.ds(d * 16, 16)] = rows_v[r, pl.ds(d * 16, 16)] * sc
            return 0

        lax.fori_loop(0, D // 16, _scale_chunk, 0)
        return 0

    lax.fori_loop(0, _ROWS_PER_SC, _scale_row, 0)
    pltpu.make_async_copy(rows_v, out_hbm.at[idx_v], gsem).start()
    pltpu.make_async_copy(rows_v, out_hbm.at[idx_v], gsem).wait()


def _sc_apply(v_flat, scores_flat, cut, out_ref, L, D):
    B = scores_flat.shape[0]
    mesh = plsc.VectorSubcoreMesh(core_axis_name="c", subcore_axis_name="s")
    body = functools.partial(_sc_body, L, D, B)
    run = pl.kernel(
        body,
        out_type=(),
        mesh=mesh,
        scratch_types=[
            pltpu.VMEM((L,), jnp.float32),
            pltpu.VMEM((16,), jnp.float32),
            pltpu.VMEM((16,), jnp.int32),
            pltpu.VMEM((TOPK,), jnp.int32),
            pltpu.VMEM((TOPK,), jnp.float32),
            pltpu.VMEM((_ROWS_PER_SC,), jnp.int32),
            pltpu.VMEM((_ROWS_PER_SC,), jnp.float32),
            pltpu.VMEM((_ROWS_PER_SC, D), jnp.float32),
            pltpu.VMEM_SHARED((2, TOPK), jnp.int32),
            pltpu.VMEM_SHARED((2, TOPK), jnp.float32),
            pltpu.SemaphoreType.DMA,
        ],
        compiler_params=pltpu.CompilerParams(needs_layout_passes=False),
    )
    run(v_flat, scores_flat, cut, out_ref)


@jax.jit
def kernel(queries, keys, values):
    B, L, D = queries.shape
    canvas, scores, cut = _tc_scores(queries, keys)
    out_ref = jax.new_ref(canvas.reshape(B * L, D))
    _sc_apply(
        values.reshape(B * L, D), scores.reshape(B, L), cut.reshape(B, 16),
        out_ref, L, D,
    )
    return out_ref[...].reshape(B, L, D)


# speedup vs baseline: 1.1175x; 1.0723x over previous
import functools

import jax
import jax.numpy as jnp
from jax import lax
from jax.experimental import pallas as pl
from jax.experimental.pallas import tpu as pltpu
from jax.experimental.pallas import tpu_sc as plsc

TOPK = 128
EPS = 0.001
LT = 8


def _tc_body(q_ref, k_ref, z_ref, s_ref, cut_ref, colsum):
    z_ref[...] = jnp.zeros_like(z_ref)
    ph = pl.program_id(1)

    @pl.when(ph == 0)
    def _colsum_phase():
        qp = jax.nn.relu(q_ref[0]) + EPS
        colsum[...] = jax.lax.dot_general(
            jnp.full((1, qp.shape[0]), 1.0, jnp.float32), qp,
            (((1,), (0,)), ((), ())),
            preferred_element_type=jnp.float32,
        )

    @pl.when(ph == 1)
    def _score_phase():
        kp = jax.nn.relu(k_ref[0]) + EPS
        sall = jax.lax.dot_general(
            colsum[...], kp, (((1,), (1,)), ((), ())),
            preferred_element_type=jnp.float32,
        )

        def step(_, lohi):
            lo, hi = lohi
            mid = lo + (hi - lo) // 2
            mid_f = jax.lax.bitcast_convert_type(mid, jnp.float32)
            cnt = jnp.sum((sall > mid_f).astype(jnp.int32))
            take = cnt <= TOPK
            return (
                jnp.where(take, lo, mid + 1),
                jnp.where(take, mid, hi),
            )

        lo, _ = jax.lax.fori_loop(
            0, 31, step, (jnp.int32(0), jnp.int32(0x7F800000))
        )
        cut_f = jax.lax.bitcast_convert_type(lo, jnp.float32)
        s_ref[0] = sall
        cut_ref[0, 0, :] = jnp.full((16,), cut_f, jnp.float32)


def _tc_scores(queries, keys):
    B, L, D = queries.shape
    blk = (1, L, D)

    def q_map(b, ph):
        return (b, 0, 0)

    def k_map(b, ph):
        return (b, 0, 0)

    def o_map(b, ph):
        return (b, 0, 0)

    def z_map(b, ph):
        return (b, ph, 0)

    return pl.pallas_call(
        _tc_body,
        grid=(B, 2),
        in_specs=[
            pl.BlockSpec(blk, q_map),
            pl.BlockSpec(blk, k_map),
        ],
        out_specs=[
            pl.BlockSpec((1, L // 2, D), z_map),
            pl.BlockSpec((1, 1, L), o_map),
            pl.BlockSpec((1, 1, 16), o_map),
        ],
        out_shape=[
            jax.ShapeDtypeStruct((B, L, D), jnp.float32),
            jax.ShapeDtypeStruct((B, 1, L), jnp.float32),
            jax.ShapeDtypeStruct((B, 1, 16), jnp.float32),
        ],
        scratch_shapes=[
            pltpu.VMEM((1, D), jnp.float32),
        ],
        compiler_params=pltpu.CompilerParams(
            dimension_semantics=("arbitrary", "arbitrary"),
        ),
    )(queries, keys)


_NTILE = 16
_ROWS_PER_SC = 16


def _take16(vec, idx):
    return lax.gather(
        vec,
        idx[:, None],
        lax.GatherDimensionNumbers(
            offset_dims=(), collapsed_slice_dims=(0,), start_index_map=(0,)
        ),
        (1,),
        mode=lax.GatherScatterMode.PROMISE_IN_BOUNDS,
    )


def _sc_body(L, D, B, v_hbm, s_hbm, cut_hbm, out_hbm,
             scbuf, cutv, cutl, idxbuf, mbuf, idx_v, m_v, rows_v,
             spm_idx, spm_m, gsem):
    c = lax.axis_index("c")
    s = lax.axis_index("s")
    batches_per_core = B // 2

    @pl.when(s < batches_per_core)
    def _compact():
        b = c * batches_per_core + s
        pltpu.sync_copy(s_hbm.at[b], scbuf)
        pltpu.sync_copy(cut_hbm.at[b], cutv)
        cutf = cutv[...]
        zeros16 = jnp.zeros((16,), jnp.int32)

        def _findcut(i, _):
            sv = scbuf[pl.ds(i * 16, 16)]
            lvec = lax.iota(jnp.int32, 16) + i * 16
            plsc.store_scatter(cutl, [zeros16], lvec, mask=sv == cutf)
            return 0

        lax.fori_loop(0, L // 16, _findcut, 0)
        pad = _take16(cutl[...], zeros16) + b * L

        for j in range(TOPK // 16):
            idxbuf[pl.ds(j * 16, 16)] = pad
            mbuf[pl.ds(j * 16, 16)] = jnp.zeros((16,), jnp.float32)

        def _scan(i, carry):
            sv = scbuf[pl.ds(i * 16, 16)]
            keep = sv > cutf
            mi = keep.astype(jnp.int32)
            incl = plsc.cumsum(mi)
            pos = incl - mi + carry
            lvec = lax.iota(jnp.int32, 16) + i * 16 + b * L
            plsc.store_scatter(idxbuf, [pos], lvec, mask=keep)
            plsc.store_scatter(
                mbuf, [pos], jnp.ones((16,), jnp.float32), mask=keep
            )
            return carry + incl[15]

        lax.fori_loop(0, L // 16, _scan, jnp.int32(0))
        pltpu.sync_copy(idxbuf, spm_idx.at[s])
        pltpu.sync_copy(mbuf, spm_m.at[s])

    plsc.subcore_barrier()

    bb = s // (_NTILE // batches_per_core)
    off = (s % (_NTILE // batches_per_core)) * _ROWS_PER_SC
    pltpu.sync_copy(spm_idx.at[bb, pl.ds(off, _ROWS_PER_SC)], idx_v)
    pltpu.sync_copy(spm_m.at[bb, pl.ds(off, _ROWS_PER_SC)], m_v)
    pltpu.make_async_copy(v_hbm.at[idx_v], rows_v, gsem).start()
    pltpu.make_async_copy(v_hbm.at[idx_v], rows_v, gsem).wait()
    mv = m_v[...]

    @pl.when(jnp.sum(mv) < 15.5)
    def _apply_pad_scales():
        def _scale_row(r, _):
            sc = _take16(mv, jnp.full((16,), r, jnp.int32))

            def _scale_chunk(d, _):
                rows_v[r, pl.ds(d * 16, 16)] = (
                    rows_v[r, pl.ds(d * 16, 16)] * sc
                )
                return 0

            lax.fori_loop(0, D // 16, _scale_chunk, 0)
            return 0

        lax.fori_loop(0, _ROWS_PER_SC, _scale_row, 0)
    pltpu.make_async_copy(rows_v, out_hbm.at[idx_v], gsem).start()
    pltpu.make_async_copy(rows_v, out_hbm.at[idx_v], gsem).wait()


def _sc_apply(v_flat, scores_flat, cut, out_ref, L, D):
    B = scores_flat.shape[0]
    mesh = plsc.VectorSubcoreMesh(core_axis_name="c", subcore_axis_name="s")
    body = functools.partial(_sc_body, L, D, B)
    run = pl.kernel(
        body,
        out_type=(),
        mesh=mesh,
        scratch_types=[
            pltpu.VMEM((L,), jnp.float32),
            pltpu.VMEM((16,), jnp.float32),
            pltpu.VMEM((16,), jnp.int32),
            pltpu.VMEM((TOPK,), jnp.int32),
            pltpu.VMEM((TOPK,), jnp.float32),
            pltpu.VMEM((_ROWS_PER_SC,), jnp.int32),
            pltpu.VMEM((_ROWS_PER_SC,), jnp.float32),
            pltpu.VMEM((_ROWS_PER_SC, D), jnp.float32),
            pltpu.VMEM_SHARED((2, TOPK), jnp.int32),
            pltpu.VMEM_SHARED((2, TOPK), jnp.float32),
            pltpu.SemaphoreType.DMA,
        ],
        compiler_params=pltpu.CompilerParams(needs_layout_passes=False),
    )
    run(v_flat, scores_flat, cut, out_ref)


@jax.jit
def kernel(queries, keys, values):
    B, L, D = queries.shape
    canvas, scores, cut = _tc_scores(queries, keys)
    out_ref = jax.new_ref(canvas.reshape(B * L, D))
    _sc_apply(
        values.reshape(B * L, D), scores.reshape(B, L), cut.reshape(B, 16),
        out_ref, L, D,
    )
    return out_ref[...].reshape(B, L, D)
